# baseline (speedup 1.0000x reference)
import functools

import jax
import jax.numpy as jnp
from jax import lax
from jax.experimental import pallas as pl
from jax.experimental.pallas import tpu as pltpu
from jax.experimental.pallas import tpu_sc as plsc

B = 1024
NI = 1000
NCAND = 100
NSEEN = 200

NCORES = 2
NSUB = 16
NW = NCORES * NSUB

CHUNKS = 2
CB = B // CHUNKS
CROWS = CB // NW

_mesh = plsc.VectorSubcoreMesh(core_axis_name="c", subcore_axis_name="s")
_sc_params = pltpu.CompilerParams(needs_layout_passes=False)


def _wid():
    return lax.axis_index("s") * NCORES + lax.axis_index("c")


def _make_build_v(chunk):
    @functools.partial(
        pl.kernel,
        mesh=_mesh,
        out_type=jax.ShapeDtypeStruct((CB, NI), jnp.float32),
        scratch_types=[
            pltpu.VMEM((CROWS, NSEEN), jnp.int32),
            pltpu.VMEM((CROWS, NCAND), jnp.int32),
            pltpu.VMEM((CROWS, NI), jnp.float32),
        ],
        compiler_params=_sc_params,
    )
    def bv(seen_hbm, item_hbm, v_hbm, seen_v, item_v, v_v):
        wid = _wid()
        src = chunk * CB + wid * CROWS
        dst = wid * CROWS
        pltpu.sync_copy(seen_hbm.at[pl.ds(src, CROWS)], seen_v)
        pltpu.sync_copy(item_hbm.at[pl.ds(src, CROWS)], item_v)

        zeros16 = jnp.zeros((16,), jnp.float32)
        ones16 = jnp.ones((16,), jnp.float32)

        @plsc.parallel_loop(0, CROWS, unroll=2)
        def _zero(r):
            for j in range(NI // 16):
                v_v[r, pl.ds(j * 16, 16)] = zeros16
            v_v[r, pl.ds(NI - 16, 16)] = zeros16

        rows = lax.iota(jnp.int32, 16)

        @plsc.parallel_loop(0, NSEEN, unroll=8)
        def _scatter_seen(s):
            col = jnp.full((16,), s, jnp.int32)
            vals = plsc.load_gather(seen_v, [rows, col])
            plsc.addupdate_scatter(v_v, [rows, vals], ones16)

        @plsc.parallel_loop(0, NCAND, unroll=8)
        def _zero_items(i):
            col = jnp.full((16,), i, jnp.int32)
            vals = plsc.load_gather(item_v, [rows, col])
            plsc.store_scatter(v_v, [rows, vals], zeros16)

        pltpu.sync_copy(v_v, v_hbm.at[pl.ds(dst, CROWS)])

    return bv


def _matmul_body(v_ref, w_ref, o_ref):
    o_ref[...] = lax.dot_general(
        v_ref[...].astype(jnp.bfloat16), w_ref[...],
        (((1,), (1,)), ((), ())),
        preferred_element_type=jnp.float32,
    )


def _matmul(v, w_bf16):
    return pl.pallas_call(
        _matmul_body,
        grid=(CB // 128,),
        in_specs=[
            pl.BlockSpec((128, NI), lambda i: (i, 0)),
            pl.BlockSpec((NI, NI), lambda i: (0, 0)),
        ],
        out_specs=pl.BlockSpec((128, NI), lambda i: (i, 0)),
        out_shape=jax.ShapeDtypeStruct((CB, NI), jnp.float32),
    )(v, w_bf16)


def _make_gather(chunk):
    @functools.partial(
        pl.kernel,
        mesh=_mesh,
        out_type=jax.ShapeDtypeStruct((CB, NCAND), jnp.float32),
        scratch_types=[
            pltpu.VMEM((CROWS, NI), jnp.float32),
            pltpu.VMEM((CROWS, NCAND), jnp.int32),
            pltpu.VMEM((CROWS, NCAND), jnp.float32),
        ],
        compiler_params=_sc_params,
    )
    def g(scores_hbm, item_hbm, out_hbm, scores_v, item_v, out_v):
        wid = _wid()
        dst = wid * CROWS
        pltpu.sync_copy(scores_hbm.at[pl.ds(dst, CROWS)], scores_v)
        pltpu.sync_copy(item_hbm.at[pl.ds(chunk * CB + dst, CROWS)], item_v)

        rows = lax.iota(jnp.int32, 16)

        @plsc.parallel_loop(0, NCAND, unroll=8)
        def _gather(i):
            col = jnp.full((16,), i, jnp.int32)
            it = plsc.load_gather(item_v, [rows, col])
            vals = plsc.load_gather(scores_v, [rows, it])
            plsc.store_scatter(out_v, [rows, col], vals)

        pltpu.sync_copy(out_v, out_hbm.at[pl.ds(dst, CROWS)])

    return g


_build_v = [_make_build_v(c) for c in range(CHUNKS)]
_gather_scores = [_make_gather(c) for c in range(CHUNKS)]


def kernel(x, item, seen_items, weights):
    w_bf16 = weights.astype(jnp.bfloat16)
    outs = []
    for c in range(CHUNKS):
        v = _build_v[c](seen_items, item)
        scores = _matmul(v, w_bf16)
        outs.append(_gather_scores[c](scores, item))
    return jnp.concatenate(outs, axis=0)


# speedup vs baseline: 1.0308x; 1.0308x over previous
import functools

import jax
import jax.numpy as jnp
from jax import lax
from jax.experimental import pallas as pl
from jax.experimental.pallas import tpu as pltpu
from jax.experimental.pallas import tpu_sc as plsc

B = 1024
NI = 1000
NCAND = 100
NSEEN = 200

NCORES = 2
NSUB = 16
NW = NCORES * NSUB

CHUNKS = 2
CB = B // CHUNKS
CROWS = CB // NW

_mesh = plsc.VectorSubcoreMesh(core_axis_name="c", subcore_axis_name="s")
_sc_params = pltpu.CompilerParams(needs_layout_passes=False)


def _wid():
    return lax.axis_index("s") * NCORES + lax.axis_index("c")


def _make_build_v(chunk):
    @functools.partial(
        pl.kernel,
        mesh=_mesh,
        out_type=jax.ShapeDtypeStruct((CB, NI), jnp.float32),
        scratch_types=[
            pltpu.VMEM((CROWS, NSEEN), jnp.int32),
            pltpu.VMEM((CROWS, NCAND), jnp.int32),
            pltpu.VMEM((CROWS, NI), jnp.float32),
        ],
        compiler_params=_sc_params,
    )
    def bv(seen_hbm, item_hbm, v_hbm, seen_v, item_v, v_v):
        wid = _wid()
        src = chunk * CB + wid * CROWS
        dst = wid * CROWS
        pltpu.sync_copy(seen_hbm.at[pl.ds(src, CROWS)], seen_v)
        pltpu.sync_copy(item_hbm.at[pl.ds(src, CROWS)], item_v)

        zeros16 = jnp.zeros((16,), jnp.float32)
        ones16 = jnp.ones((16,), jnp.float32)

        @plsc.parallel_loop(0, CROWS, unroll=2)
        def _zero(r):
            for j in range(NI // 16):
                v_v[r, pl.ds(j * 16, 16)] = zeros16
            v_v[r, pl.ds(NI - 16, 16)] = zeros16

        rows = lax.iota(jnp.int32, 16)

        @plsc.parallel_loop(0, NSEEN, unroll=8)
        def _scatter_seen(s):
            col = jnp.full((16,), s, jnp.int32)
            vals = plsc.load_gather(seen_v, [rows, col])
            plsc.addupdate_scatter(v_v, [rows, vals], ones16)

        @plsc.parallel_loop(0, NCAND, unroll=8)
        def _zero_items(i):
            col = jnp.full((16,), i, jnp.int32)
            vals = plsc.load_gather(item_v, [rows, col])
            plsc.store_scatter(v_v, [rows, vals], zeros16)

        pltpu.sync_copy(v_v, v_hbm.at[pl.ds(dst, CROWS)])

    return bv


def _matmul_body(v_ref, w_ref, o_ref):
    o_ref[...] = lax.dot_general(
        v_ref[...].astype(jnp.bfloat16), w_ref[...],
        (((1,), (1,)), ((), ())),
        preferred_element_type=jnp.float32,
    )


def _matmul(v, w_bf16):
    return pl.pallas_call(
        _matmul_body,
        grid=(CB // 128,),
        in_specs=[
            pl.BlockSpec((128, NI), lambda i: (i, 0)),
            pl.BlockSpec((NI, NI), lambda i: (0, 0)),
        ],
        out_specs=pl.BlockSpec((128, NI), lambda i: (i, 0)),
        out_shape=jax.ShapeDtypeStruct((CB, NI), jnp.float32),
    )(v, w_bf16)


GROWS = B // NW


@functools.partial(
    pl.kernel,
    mesh=_mesh,
    out_type=jax.ShapeDtypeStruct((B, NCAND), jnp.float32),
    scratch_types=[
        pltpu.VMEM((GROWS, NI), jnp.float32),
        pltpu.VMEM((GROWS, NCAND), jnp.int32),
        pltpu.VMEM((GROWS, NCAND), jnp.float32),
    ],
    compiler_params=_sc_params,
)
def _gather_scores(s0_hbm, s1_hbm, item_hbm, out_hbm, scores_v, item_v, out_v):
    wid = _wid()
    dst = wid * GROWS

    @pl.when(wid < NW // 2)
    def _():
        pltpu.sync_copy(s0_hbm.at[pl.ds(dst, GROWS)], scores_v)

    @pl.when(wid >= NW // 2)
    def _():
        pltpu.sync_copy(s1_hbm.at[pl.ds(dst - CB, GROWS)], scores_v)

    pltpu.sync_copy(item_hbm.at[pl.ds(dst, GROWS)], item_v)

    lanes = lax.iota(jnp.int32, 16)
    row_groups = [lanes + g * 16 for g in range(GROWS // 16)]

    @plsc.parallel_loop(0, NCAND, unroll=8)
    def _gather(i):
        col = jnp.full((16,), i, jnp.int32)
        for rows in row_groups:
            it = plsc.load_gather(item_v, [rows, col])
            vals = plsc.load_gather(scores_v, [rows, it])
            plsc.store_scatter(out_v, [rows, col], vals)

    pltpu.sync_copy(out_v, out_hbm.at[pl.ds(dst, GROWS)])


_build_v = [_make_build_v(c) for c in range(CHUNKS)]


def kernel(x, item, seen_items, weights):
    w_bf16 = weights.astype(jnp.bfloat16)
    scores = [_matmul(_build_v[c](seen_items, item), w_bf16)
              for c in range(CHUNKS)]
    return _gather_scores(scores[0], scores[1], item)


# speedup vs baseline: 1.0718x; 1.0398x over previous
import functools

import jax
import jax.numpy as jnp
from jax import lax
from jax.experimental import pallas as pl
from jax.experimental.pallas import tpu as pltpu
from jax.experimental.pallas import tpu_sc as plsc

B = 1024
NI = 1000
NCAND = 100
NSEEN = 200

NCORES = 2
NSUB = 16
NW = NCORES * NSUB
ROWS = B // NW

_mesh = plsc.VectorSubcoreMesh(core_axis_name="c", subcore_axis_name="s")
_sc_params = pltpu.CompilerParams(needs_layout_passes=False)


def _wid():
    return lax.axis_index("s") * NCORES + lax.axis_index("c")


@functools.partial(
    pl.kernel,
    mesh=_mesh,
    out_type=jax.ShapeDtypeStruct((B, NI), jnp.float32),
    scratch_types=[
        pltpu.VMEM((ROWS, NSEEN), jnp.int32),
        pltpu.VMEM((ROWS, NCAND), jnp.int32),
        pltpu.VMEM((ROWS, NI), jnp.float32),
    ],
    compiler_params=_sc_params,
)
def _build_v(seen_hbm, item_hbm, v_hbm, seen_v, item_v, v_v):
    base = _wid() * ROWS
    pltpu.sync_copy(seen_hbm.at[pl.ds(base, ROWS)], seen_v)
    pltpu.sync_copy(item_hbm.at[pl.ds(base, ROWS)], item_v)

    zeros16 = jnp.zeros((16,), jnp.float32)
    ones16 = jnp.ones((16,), jnp.float32)

    @plsc.parallel_loop(0, ROWS, unroll=2)
    def _zero(r):
        for j in range(NI // 16):
            v_v[r, pl.ds(j * 16, 16)] = zeros16
        v_v[r, pl.ds(NI - 16, 16)] = zeros16

    lanes = lax.iota(jnp.int32, 16)
    row_groups = [lanes + g * 16 for g in range(ROWS // 16)]

    @plsc.parallel_loop(0, NSEEN, unroll=8)
    def _scatter_seen(s):
        col = jnp.full((16,), s, jnp.int32)
        for rows in row_groups:
            vals = plsc.load_gather(seen_v, [rows, col])
            plsc.addupdate_scatter(v_v, [rows, vals], ones16)

    @plsc.parallel_loop(0, NCAND, unroll=8)
    def _zero_items(i):
        col = jnp.full((16,), i, jnp.int32)
        for rows in row_groups:
            vals = plsc.load_gather(item_v, [rows, col])
            plsc.store_scatter(v_v, [rows, vals], zeros16)

    pltpu.sync_copy(v_v, v_hbm.at[pl.ds(base, ROWS)])


def _matmul_body(v_ref, w_ref, o_ref):
    o_ref[...] = lax.dot_general(
        v_ref[...], w_ref[...],
        (((1,), (1,)), ((), ())),
        preferred_element_type=jnp.float32,
    )


def _matmul(v, weights):
    return pl.pallas_call(
        _matmul_body,
        grid=(8,),
        in_specs=[
            pl.BlockSpec((B // 8, NI), lambda i: (i, 0)),
            pl.BlockSpec((NI, NI), lambda i: (0, 0)),
        ],
        out_specs=pl.BlockSpec((B // 8, NI), lambda i: (i, 0)),
        out_shape=jax.ShapeDtypeStruct((B, NI), jnp.float32),
    )(v, weights)


@functools.partial(
    pl.kernel,
    mesh=_mesh,
    out_type=jax.ShapeDtypeStruct((B, NCAND), jnp.float32),
    scratch_types=[
        pltpu.VMEM((ROWS, NI), jnp.float32),
        pltpu.VMEM((ROWS, NCAND), jnp.int32),
        pltpu.VMEM((ROWS, NCAND), jnp.float32),
    ],
    compiler_params=_sc_params,
)
def _gather_scores(scores_hbm, item_hbm, out_hbm, scores_v, item_v, out_v):
    base = _wid() * ROWS
    pltpu.sync_copy(scores_hbm.at[pl.ds(base, ROWS)], scores_v)
    pltpu.sync_copy(item_hbm.at[pl.ds(base, ROWS)], item_v)

    lanes = lax.iota(jnp.int32, 16)
    row_groups = [lanes + g * 16 for g in range(ROWS // 16)]

    @plsc.parallel_loop(0, NCAND, unroll=8)
    def _gather(i):
        col = jnp.full((16,), i, jnp.int32)
        for rows in row_groups:
            it = plsc.load_gather(item_v, [rows, col])
            vals = plsc.load_gather(scores_v, [rows, it])
            plsc.store_scatter(out_v, [rows, col], vals)

    pltpu.sync_copy(out_v, out_hbm.at[pl.ds(base, ROWS)])


def kernel(x, item, seen_items, weights):
    v = _build_v(seen_items, item)
    scores = _matmul(v, weights)
    return _gather_scores(scores, item)


# speedup vs baseline: 1.0735x; 1.0015x over previous
import functools

import jax
import jax.numpy as jnp
from jax import lax
from jax.experimental import pallas as pl
from jax.experimental.pallas import tpu as pltpu
from jax.experimental.pallas import tpu_sc as plsc

B = 1024
NI = 1000
NCAND = 100
NSEEN = 200

NCORES = 2
NSUB = 16
NW = NCORES * NSUB
ROWS = B // NW

_mesh = plsc.VectorSubcoreMesh(core_axis_name="c", subcore_axis_name="s")
_sc_params = pltpu.CompilerParams(needs_layout_passes=False)


def _wid():
    return lax.axis_index("s") * NCORES + lax.axis_index("c")


@functools.partial(
    pl.kernel,
    mesh=_mesh,
    out_type=jax.ShapeDtypeStruct((B, NI), jnp.float32),
    scratch_types=[
        pltpu.VMEM((ROWS, NSEEN), jnp.int32),
        pltpu.VMEM((ROWS, NCAND), jnp.int32),
        pltpu.VMEM((ROWS, NI), jnp.float32),
    ],
    compiler_params=_sc_params,
)
def _build_v(seen_hbm, item_hbm, v_hbm, seen_v, item_v, v_v):
    base = _wid() * ROWS
    pltpu.sync_copy(seen_hbm.at[pl.ds(base, ROWS)], seen_v)
    pltpu.sync_copy(item_hbm.at[pl.ds(base, ROWS)], item_v)

    zeros16 = jnp.zeros((16,), jnp.float32)
    ones16 = jnp.ones((16,), jnp.float32)

    @plsc.parallel_loop(0, ROWS, unroll=2)
    def _zero(r):
        for j in range(NI // 16):
            v_v[r, pl.ds(j * 16, 16)] = zeros16
        v_v[r, pl.ds(NI - 16, 16)] = zeros16

    lanes = lax.iota(jnp.int32, 16)
    row_groups = [lanes + g * 16 for g in range(ROWS // 16)]

    @plsc.parallel_loop(0, NSEEN, unroll=16)
    def _scatter_seen(s):
        col = jnp.full((16,), s, jnp.int32)
        for rows in row_groups:
            vals = plsc.load_gather(seen_v, [rows, col])
            plsc.addupdate_scatter(v_v, [rows, vals], ones16)

    @plsc.parallel_loop(0, NCAND, unroll=10)
    def _zero_items(i):
        col = jnp.full((16,), i, jnp.int32)
        for rows in row_groups:
            vals = plsc.load_gather(item_v, [rows, col])
            plsc.store_scatter(v_v, [rows, vals], zeros16)

    pltpu.sync_copy(v_v, v_hbm.at[pl.ds(base, ROWS)])


def _matmul_body(v_ref, w_ref, o_ref):
    o_ref[...] = lax.dot_general(
        v_ref[...], w_ref[...],
        (((1,), (1,)), ((), ())),
        preferred_element_type=jnp.float32,
    )


def _matmul(v, weights):
    return pl.pallas_call(
        _matmul_body,
        grid=(8,),
        in_specs=[
            pl.BlockSpec((B // 8, NI), lambda i: (i, 0)),
            pl.BlockSpec((NI, NI), lambda i: (0, 0)),
        ],
        out_specs=pl.BlockSpec((B // 8, NI), lambda i: (i, 0)),
        out_shape=jax.ShapeDtypeStruct((B, NI), jnp.float32),
    )(v, weights)


@functools.partial(
    pl.kernel,
    mesh=_mesh,
    out_type=jax.ShapeDtypeStruct((B, NCAND), jnp.float32),
    scratch_types=[
        pltpu.VMEM((ROWS, NI), jnp.float32),
        pltpu.VMEM((ROWS, NCAND), jnp.int32),
        pltpu.VMEM((ROWS, NCAND), jnp.float32),
    ],
    compiler_params=_sc_params,
)
def _gather_scores(scores_hbm, item_hbm, out_hbm, scores_v, item_v, out_v):
    base = _wid() * ROWS
    pltpu.sync_copy(scores_hbm.at[pl.ds(base, ROWS)], scores_v)
    pltpu.sync_copy(item_hbm.at[pl.ds(base, ROWS)], item_v)

    lanes = lax.iota(jnp.int32, 16)
    row_groups = [lanes + g * 16 for g in range(ROWS // 16)]

    @plsc.parallel_loop(0, NCAND, unroll=10)
    def _gather(i):
        col = jnp.full((16,), i, jnp.int32)
        for rows in row_groups:
            it = plsc.load_gather(item_v, [rows, col])
            vals = plsc.load_gather(scores_v, [rows, it])
            plsc.store_scatter(out_v, [rows, col], vals)

    pltpu.sync_copy(out_v, out_hbm.at[pl.ds(base, ROWS)])


def kernel(x, item, seen_items, weights):
    v = _build_v(seen_items, item)
    scores = _matmul(v, weights)
    return _gather_scores(scores, item)
